# baseline (speedup 1.0000x reference)
import functools

import jax
import jax.numpy as jnp
from jax import lax
from jax.experimental import pallas as pl
from jax.experimental.pallas import tpu as pltpu
from jax.experimental.pallas import tpu_sc as plsc

_NC = 2
_NS = 16
_NW = _NC * _NS

_W = 4
_NBUF = 2


def _split_widths(n):
    widths = []
    while n > 0:
        w = min(128, n)
        widths.append(w)
        n -= w
    return widths


@functools.cache
def _build(batch, hist, vocab, d, dtype):
    rows_per_w = batch // _NW
    n_steps = rows_per_w // _W
    assert n_steps % _NBUF == 0
    pairs = _W * hist // 2
    widths = _split_widths(pairs)
    offs = [sum(widths[:j]) for j in range(len(widths))]
    assert all(o % 8 == 0 for o in offs)

    mesh = plsc.VectorSubcoreMesh(
        core_axis_name="c", subcore_axis_name="s",
        num_cores=_NC, num_subcores=_NS,
    )

    @functools.partial(
        pl.kernel,
        out_type=jax.ShapeDtypeStruct((batch * hist * d // 128, 128), dtype),
        mesh=mesh,
        scratch_types=[
            pltpu.VMEM((_NBUF, pairs), jnp.int32),
            pltpu.VMEM((_NBUF, pairs), jnp.int32),
            pltpu.VMEM((_NBUF, pairs, d), dtype),
            pltpu.VMEM((_NBUF, pairs, d), dtype),
            [pltpu.SemaphoreType.DMA] * _NBUF,
            [pltpu.SemaphoreType.DMA] * _NBUF,
            [pltpu.SemaphoreType.DMA] * _NBUF,
        ],
        compiler_params=pltpu.CompilerParams(use_tc_tiling_on_sc=False),
    )
    def gather(ie_hbm, io_hbm, table_hbm, out_hbm, ie_v, io_v, re_v, ro_v,
               isems, gsems, osems):
        wid = lax.axis_index("s") * _NC + lax.axis_index("c")
        b0 = wid * rows_per_w

        def gather_refs(b, j, half):
            idx_v = ie_v if half == 0 else io_v
            rows_v = re_v if half == 0 else ro_v
            src = table_hbm.at[idx_v.at[b].at[pl.ds(offs[j], widths[j])]]
            dst = rows_v.at[b].at[pl.ds(offs[j], widths[j])]
            return src, dst

        def fire_gathers(b):
            for j in range(len(widths)):
                for half in (0, 1):
                    src, dst = gather_refs(b, j, half)
                    pltpu.async_copy(src, dst, gsems[b])

        def wait_gathers(b):
            for j in range(len(widths)):
                for half in (0, 1):
                    src, dst = gather_refs(b, j, half)
                    pltpu.make_async_copy(src, dst, gsems[b]).wait()

        def writeback_refs(b, rb, half):
            src = (re_v if half == 0 else ro_v).at[b]
            dst = out_hbm.at[pl.ds(pl.multiple_of(rb * hist // 2, 8), pairs),
                             pl.ds(half * d, d)]
            return src, dst

        def fire_writeback(b, rb):
            for half in (0, 1):
                src, dst = writeback_refs(b, rb, half)
                pltpu.async_copy(src, dst, osems[b])

        def wait_writeback(b, rb):
            for half in (0, 1):
                src, dst = writeback_refs(b, rb, half)
                pltpu.make_async_copy(src, dst, osems[b]).wait()

        def fire_idx(b, rb):
            o = pl.multiple_of(rb * hist // 2, 8)
            pltpu.async_copy(ie_hbm.at[pl.ds(o, pairs)], ie_v.at[b], isems[b])
            pltpu.async_copy(io_hbm.at[pl.ds(o, pairs)], io_v.at[b], isems[b])

        def wait_idx(b, rb):
            o = pl.multiple_of(rb * hist // 2, 8)
            pltpu.make_async_copy(
                ie_hbm.at[pl.ds(o, pairs)], ie_v.at[b], isems[b]).wait()
            pltpu.make_async_copy(
                io_hbm.at[pl.ds(o, pairs)], io_v.at[b], isems[b]).wait()

        for b in range(_NBUF):
            fire_idx(b, b0 + b * _W)

        @pl.loop(0, n_steps, step=_NBUF)
        def _step(g0):
            for b in range(_NBUF):
                g = g0 + b
                rb = b0 + g * _W
                p = (b - 1) % _NBUF

                @pl.when(g0 >= _NBUF)
                def _():
                    wait_writeback(b, rb)

                wait_idx(b, rb)

                fire_gathers(b)

                @pl.when(g >= 1)
                def _():
                    wait_gathers(p)
                    fire_writeback(p, rb - _W)

                    @pl.when(g - 1 + _NBUF < n_steps)
                    def _():
                        fire_idx(p, rb + (_NBUF - 1) * _W)

        last = (n_steps - 1) % _NBUF
        rb_last = b0 + (n_steps - 1) * _W
        wait_gathers(last)
        fire_writeback(last, rb_last)
        for b in range(_NBUF):
            wait_writeback(b, b0)

    return gather


def kernel(x, table):
    batch, hist = x.shape
    vocab, d = table.shape
    flat2 = x.reshape(-1, 2).astype(jnp.int32)
    flat_e = flat2[:, 0]
    flat_o = flat2[:, 1]
    assert batch % (_NW * _W) == 0 and hist % 8 == 0 and (2 * d) == 128
    out2d = _build(batch, hist, vocab, d, table.dtype)(flat_e, flat_o, table)
    return out2d.reshape(batch, hist, d)


# speedup vs baseline: 1.6099x; 1.6099x over previous
import functools

import jax
import jax.numpy as jnp
from jax import lax
from jax.experimental import pallas as pl
from jax.experimental.pallas import tpu as pltpu
from jax.experimental.pallas import tpu_sc as plsc

_NC = 2
_NS = 16
_NW = _NC * _NS

_W = 4
_NBUF = 2


def _split_widths(n):
    widths = []
    while n > 0:
        w = min(128, n)
        widths.append(w)
        n -= w
    return widths


@functools.cache
def _build(batch, hist, vocab, d, dtype):
    rows_per_w = batch // _NW
    n_steps = rows_per_w // _W
    assert n_steps % _NBUF == 0
    hh = hist // 2
    pairs = _W * hh
    widths = _split_widths(pairs)
    offs = [sum(widths[:j]) for j in range(len(widths))]
    assert all(o % 8 == 0 for o in offs)

    mesh = plsc.VectorSubcoreMesh(
        core_axis_name="c", subcore_axis_name="s",
        num_cores=_NC, num_subcores=_NS,
    )

    @functools.partial(
        pl.kernel,
        out_type=jax.ShapeDtypeStruct((hh, batch, 2 * d), dtype),
        mesh=mesh,
        scratch_types=[
            pltpu.VMEM((_NBUF, pairs), jnp.int32),
            pltpu.VMEM((_NBUF, pairs), jnp.int32),
            pltpu.VMEM((_NBUF, pairs, d), dtype),
            pltpu.VMEM((_NBUF, pairs, d), dtype),
            [pltpu.SemaphoreType.DMA] * _NBUF,
            [pltpu.SemaphoreType.DMA] * _NBUF,
            [pltpu.SemaphoreType.DMA] * _NBUF,
        ],
        compiler_params=pltpu.CompilerParams(use_tc_tiling_on_sc=False),
    )
    def gather(ie_hbm, io_hbm, table_hbm, out_hbm, ie_v, io_v, re_v, ro_v,
               isems, gsems, osems):
        wid = lax.axis_index("s") * _NC + lax.axis_index("c")
        b0 = wid * rows_per_w

        def gather_refs(b, j, half):
            idx_v = ie_v if half == 0 else io_v
            rows_v = re_v if half == 0 else ro_v
            src = table_hbm.at[idx_v.at[b].at[pl.ds(offs[j], widths[j])]]
            dst = rows_v.at[b].at[pl.ds(offs[j], widths[j])]
            return src, dst

        def fire_gathers(b):
            for j in range(len(widths)):
                for half in (0, 1):
                    src, dst = gather_refs(b, j, half)
                    pltpu.async_copy(src, dst, gsems[b])

        def wait_gathers(b):
            for j in range(len(widths)):
                for half in (0, 1):
                    src, dst = gather_refs(b, j, half)
                    pltpu.make_async_copy(src, dst, gsems[b]).wait()

        def writeback_refs(b, rb, i, half):
            src = (re_v if half == 0 else ro_v).at[b].at[pl.ds(i * hh, hh)]
            dst = out_hbm.at[:, rb + i, pl.ds(half * d, d)]
            return src, dst

        def fire_writebacks(b, rb):
            for i in range(_W):
                for half in (0, 1):
                    src, dst = writeback_refs(b, rb, i, half)
                    pltpu.async_copy(src, dst, osems[b])

        def wait_writebacks(b, rb):
            for i in range(_W):
                for half in (0, 1):
                    src, dst = writeback_refs(b, rb, i, half)
                    pltpu.make_async_copy(src, dst, osems[b]).wait()

        def fire_idx(b, rb):
            o = pl.multiple_of(rb * hh, 8)
            pltpu.async_copy(ie_hbm.at[pl.ds(o, pairs)], ie_v.at[b], isems[b])
            pltpu.async_copy(io_hbm.at[pl.ds(o, pairs)], io_v.at[b], isems[b])

        def wait_idx(b, rb):
            o = pl.multiple_of(rb * hh, 8)
            pltpu.make_async_copy(
                ie_hbm.at[pl.ds(o, pairs)], ie_v.at[b], isems[b]).wait()
            pltpu.make_async_copy(
                io_hbm.at[pl.ds(o, pairs)], io_v.at[b], isems[b]).wait()

        for b in range(_NBUF):
            fire_idx(b, b0 + b * _W)

        @pl.loop(0, n_steps, step=_NBUF)
        def _step(g0):
            for b in range(_NBUF):
                g = g0 + b
                rb = b0 + g * _W
                p = (b - 1) % _NBUF

                @pl.when(g0 >= _NBUF)
                def _():
                    wait_writebacks(b, rb)

                wait_idx(b, rb)

                fire_gathers(b)

                @pl.when(g >= 1)
                def _():
                    wait_gathers(p)
                    fire_writebacks(p, rb - _W)

                    @pl.when(g - 1 + _NBUF < n_steps)
                    def _():
                        fire_idx(p, rb + (_NBUF - 1) * _W)

        last = (n_steps - 1) % _NBUF
        rb_last = b0 + (n_steps - 1) * _W
        wait_gathers(last)
        fire_writebacks(last, rb_last)
        for b in range(_NBUF):
            wait_writebacks(b, b0)

    return gather


@functools.cache
def _build_transpose(batch, hist, d, dtype):
    hh = hist // 2
    bt = 128

    def body(x_ref, p_ref):
        for kk in range(hh):
            y = x_ref[kk].T
            p_ref[2 * kk] = y[:d]
            p_ref[2 * kk + 1] = y[d:]

    return pl.pallas_call(
        body,
        grid=(batch // bt,),
        in_specs=[pl.BlockSpec((hh, bt, 2 * d), lambda i: (0, i, 0))],
        out_specs=pl.BlockSpec((hist, d, bt), lambda i: (0, 0, i)),
        out_shape=jax.ShapeDtypeStruct((hist, d, batch), dtype),
    )


def kernel(x, table):
    batch, hist = x.shape
    vocab, d = table.shape
    flat2 = x.reshape(-1, 2).astype(jnp.int32)
    flat_e = flat2[:, 0]
    flat_o = flat2[:, 1]
    assert batch % (_NW * _W) == 0 and hist % 8 == 0 and (2 * d) == 128
    out_sc = _build(batch, hist, vocab, d, table.dtype)(flat_e, flat_o, table)
    p = _build_transpose(batch, hist, d, table.dtype)(out_sc)
    return jnp.transpose(p, (2, 0, 1))


# speedup vs baseline: 1.6308x; 1.0130x over previous
import functools

import jax
import jax.numpy as jnp
from jax import lax
from jax.experimental import pallas as pl
from jax.experimental.pallas import tpu as pltpu
from jax.experimental.pallas import tpu_sc as plsc

_NC = 2
_NS = 16
_NW = _NC * _NS

_W = 4
_NBUF = 2


def _split_widths(n):
    widths = []
    while n > 0:
        w = min(128, n)
        widths.append(w)
        n -= w
    return widths


@functools.cache
def _build(batch, hist, vocab, d, dtype):
    rows_per_w = batch // _NW
    n_steps = rows_per_w // _W
    assert n_steps % _NBUF == 0
    hh = hist // 2
    pairs = _W * hh
    widths = _split_widths(pairs)
    offs = [sum(widths[:j]) for j in range(len(widths))]
    assert all(o % 8 == 0 for o in offs)

    mesh = plsc.VectorSubcoreMesh(
        core_axis_name="c", subcore_axis_name="s",
        num_cores=_NC, num_subcores=_NS,
    )

    @functools.partial(
        pl.kernel,
        out_type=jax.ShapeDtypeStruct((hh, batch, 2 * d), dtype),
        mesh=mesh,
        scratch_types=[
            pltpu.VMEM((_NBUF, pairs), jnp.int32),
            pltpu.VMEM((_NBUF, pairs), jnp.int32),
            pltpu.VMEM((_NBUF, pairs, d), dtype),
            pltpu.VMEM((_NBUF, pairs, d), dtype),
            [pltpu.SemaphoreType.DMA] * _NBUF,
            [pltpu.SemaphoreType.DMA] * _NBUF,
            [pltpu.SemaphoreType.DMA] * _NBUF,
        ],
        compiler_params=pltpu.CompilerParams(use_tc_tiling_on_sc=False),
    )
    def gather(ie_hbm, io_hbm, table_hbm, out_hbm, ie_v, io_v, re_v, ro_v,
               isems, gsems, osems):
        wid = lax.axis_index("s") * _NC + lax.axis_index("c")
        b0 = wid * rows_per_w

        def gather_refs(b, j, half):
            idx_v = ie_v if half == 0 else io_v
            rows_v = re_v if half == 0 else ro_v
            src = table_hbm.at[idx_v.at[b].at[pl.ds(offs[j], widths[j])]]
            dst = rows_v.at[b].at[pl.ds(offs[j], widths[j])]
            return src, dst

        def fire_gathers(b):
            for j in range(len(widths)):
                for half in (0, 1):
                    src, dst = gather_refs(b, j, half)
                    pltpu.async_copy(src, dst, gsems[b])

        def wait_gathers(b):
            for j in range(len(widths)):
                for half in (0, 1):
                    src, dst = gather_refs(b, j, half)
                    pltpu.make_async_copy(src, dst, gsems[b]).wait()

        def writeback_refs(b, rb, i, half):
            src = (re_v if half == 0 else ro_v).at[b].at[pl.ds(i * hh, hh)]
            dst = out_hbm.at[:, rb + i, pl.ds(half * d, d)]
            return src, dst

        def fire_writebacks(b, rb):
            for i in range(_W):
                for half in (0, 1):
                    src, dst = writeback_refs(b, rb, i, half)
                    pltpu.async_copy(src, dst, osems[b])

        def wait_writebacks(b, rb):
            for i in range(_W):
                for half in (0, 1):
                    src, dst = writeback_refs(b, rb, i, half)
                    pltpu.make_async_copy(src, dst, osems[b]).wait()

        def fire_idx(b, rb):
            o = pl.multiple_of(rb * hh, 8)
            pltpu.async_copy(ie_hbm.at[pl.ds(o, pairs)], ie_v.at[b], isems[b])
            pltpu.async_copy(io_hbm.at[pl.ds(o, pairs)], io_v.at[b], isems[b])

        def wait_idx(b, rb):
            o = pl.multiple_of(rb * hh, 8)
            pltpu.make_async_copy(
                ie_hbm.at[pl.ds(o, pairs)], ie_v.at[b], isems[b]).wait()
            pltpu.make_async_copy(
                io_hbm.at[pl.ds(o, pairs)], io_v.at[b], isems[b]).wait()

        for b in range(_NBUF):
            fire_idx(b, b0 + b * _W)

        @pl.loop(0, n_steps, step=_NBUF)
        def _step(g0):
            for b in range(_NBUF):
                g = g0 + b
                rb = b0 + g * _W
                p = (b - 1) % _NBUF

                @pl.when(g0 >= _NBUF)
                def _():
                    wait_writebacks(b, rb)

                wait_idx(b, rb)

                fire_gathers(b)

                @pl.when(g >= 1)
                def _():
                    wait_gathers(p)
                    fire_writebacks(p, rb - _W)

                    @pl.when(g - 1 + _NBUF < n_steps)
                    def _():
                        fire_idx(p, rb + (_NBUF - 1) * _W)

        last = (n_steps - 1) % _NBUF
        rb_last = b0 + (n_steps - 1) * _W
        wait_gathers(last)
        fire_writebacks(last, rb_last)
        for b in range(_NBUF):
            wait_writebacks(b, b0)

    return gather


@functools.cache
def _build_transpose(batch, hist, d, dtype, hpart, part):
    hh = hpart // 2
    bt = 128

    def body(*refs):
        x_ref, p_ref = refs[0], refs[-1]
        for kk in range(hh):
            y = x_ref[kk].T
            p_ref[2 * kk] = y[:d]
            p_ref[2 * kk + 1] = y[d:]

    in_specs = [pl.BlockSpec((hh, bt, 2 * d), lambda i: (0, i, 0))]
    kwargs = {}
    if part > 0:
        in_specs.append(pl.BlockSpec((8, 8, 128), lambda i: (0, 0, 0)))
        kwargs["input_output_aliases"] = {1: 0}
    return pl.pallas_call(
        body,
        grid=(batch // bt,),
        in_specs=in_specs,
        out_specs=pl.BlockSpec((hpart, d, bt), lambda i: (part, 0, i)),
        out_shape=jax.ShapeDtypeStruct((hist, d, batch), dtype),
        **kwargs,
    )


def _split_even_odd(xpart):
    flat2 = xpart.reshape(-1, 2).astype(jnp.int32)
    return flat2[:, 0], flat2[:, 1]


def kernel(x, table):
    batch, hist = x.shape
    vocab, d = table.shape
    assert batch % (_NW * _W) == 0 and hist % 4 == 0 and (2 * d) == 128
    hp = hist // 2
    fe1, fo1 = _split_even_odd(x[:, :hp])
    fe2, fo2 = _split_even_odd(x[:, hp:])
    sc = _build(batch, hp, vocab, d, table.dtype)
    s1 = sc(fe1, fo1, table)
    s2 = sc(fe2, fo2, table)
    p1 = _build_transpose(batch, hist, d, table.dtype, hp, 0)(s1)
    p = _build_transpose(batch, hist, d, table.dtype, hp, 1)(s2, p1)
    return jnp.transpose(p, (2, 0, 1))
